# baseline (speedup 1.0000x reference)
import functools

import jax
import jax.numpy as jnp
from jax import lax
from jax.experimental import pallas as pl
from jax.experimental.pallas import tpu as pltpu
from jax.experimental.pallas import tpu_sc as plsc

N_NODES = 10000
D_FEAT = 256
N_EDGES = 160000

NC = 2
NS = 16
DH = D_FEAT // NC
B = 128
NCHUNKS = N_EDGES // B
NCHK = NCHUNKS // NS
NBUF = 2
NBUFI = 6
LAI = 4
ROWS_PER_TILE = 624
ROWS_TAIL = N_NODES - NS * ROWS_PER_TILE
ROWS_Z = 208


def _mp_body(x, edge_index, zeros, out,
             acc, eidx, rows, gsem, ssem, isem):
    c = lax.axis_index("c")
    s = lax.axis_index("s")
    col0 = c * DH
    n_j = jnp.where(s < NCHUNKS - NS * NCHK, NCHK + 1, NCHK)

    def start_idx(j):
        bi = lax.rem(j, NBUFI)
        st = (s + NS * j) * B
        pltpu.async_copy(edge_index.at[pl.ds(0, 2), pl.ds(st, B)],
                         eidx.at[bi], isem.at[bi])

    def wait_idx(j):
        bi = lax.rem(j, NBUFI)
        st = (s + NS * j) * B
        pltpu.make_async_copy(edge_index.at[pl.ds(0, 2), pl.ds(st, B)],
                              eidx.at[bi], isem.at[bi]).wait()

    def start_gather(j, b):
        bi = lax.rem(j, NBUFI)
        pltpu.async_copy(x.at[eidx.at[bi, 1], pl.ds(col0, DH)], rows.at[b],
                         gsem.at[b])

    def wait_gather(j, b):
        bi = lax.rem(j, NBUFI)
        pltpu.make_async_copy(x.at[eidx.at[bi, 1], pl.ds(col0, DH)],
                              rows.at[b], gsem.at[b]).wait()

    def start_scatter(j, b):
        bi = lax.rem(j, NBUFI)
        pltpu.async_copy(rows.at[b], acc.at[eidx.at[bi, 0]], ssem.at[b],
                         add=True)

    def wait_scatter(j, b):
        bi = lax.rem(j, NBUFI)
        pltpu.make_async_copy(rows.at[b], acc.at[eidx.at[bi, 0]],
                              ssem.at[b]).wait()

    for k in range(LAI):
        start_idx(k)
    wait_idx(0)
    start_gather(0, 0)

    row0 = s * ROWS_PER_TILE
    for r in range(ROWS_PER_TILE // ROWS_Z):
        pltpu.sync_copy(zeros, acc.at[pl.ds(row0 + r * ROWS_Z, ROWS_Z)])

    @pl.when(s == 0)
    def _zero_tail():
        pltpu.sync_copy(zeros.at[pl.ds(0, ROWS_TAIL)],
                        acc.at[pl.ds(NS * ROWS_PER_TILE, ROWS_TAIL)])

    plsc.subcore_barrier()

    def chunk(j, carry):
        b = lax.rem(j, NBUF)
        wait_gather(j, b)
        start_scatter(j, b)

        @pl.when(j >= 1)
        def _drain():
            wait_scatter(j - 1, lax.rem(j - 1, NBUF))

        @pl.when(j + LAI < n_j)
        def _pf_idx():
            start_idx(j + LAI)

        @pl.when(j + 1 < n_j)
        def _pf_gather():
            wait_idx(j + 1)
            start_gather(j + 1, lax.rem(j + 1, NBUF))

        return carry

    lax.fori_loop(0, n_j, chunk, 0)

    wait_scatter(n_j - 1, lax.rem(n_j - 1, NBUF))

    plsc.subcore_barrier()
    pltpu.sync_copy(acc.at[pl.ds(row0, ROWS_PER_TILE)],
                    out.at[pl.ds(row0, ROWS_PER_TILE), pl.ds(col0, DH)])

    @pl.when(s == 0)
    def _out_tail():
        pltpu.sync_copy(
            acc.at[pl.ds(NS * ROWS_PER_TILE, ROWS_TAIL)],
            out.at[pl.ds(NS * ROWS_PER_TILE, ROWS_TAIL), pl.ds(col0, DH)])


_mp_call = functools.partial(
    pl.kernel,
    out_type=jax.ShapeDtypeStruct((N_NODES, D_FEAT), jnp.float32),
    mesh=plsc.VectorSubcoreMesh(core_axis_name="c", subcore_axis_name="s",
                                num_cores=NC, num_subcores=NS),
    scratch_types=[
        pltpu.VMEM_SHARED((N_NODES, DH), jnp.float32),
        pltpu.VMEM((NBUFI, 2, B), jnp.int32),
        pltpu.VMEM((NBUF, B, DH), jnp.float32),
        pltpu.SemaphoreType.DMA((NBUF,)),
        pltpu.SemaphoreType.DMA((NBUF,)),
        pltpu.SemaphoreType.DMA((NBUFI,)),
    ],
)(_mp_body)


def kernel(x, edge_index):
    ei = edge_index.astype(jnp.int32)
    zeros = jnp.zeros((ROWS_Z, DH), jnp.float32)
    return _mp_call(x, ei, zeros)


# speedup vs baseline: 1.1548x; 1.1548x over previous
import functools

import jax
import jax.numpy as jnp
from jax import lax
from jax.experimental import pallas as pl
from jax.experimental.pallas import tpu as pltpu
from jax.experimental.pallas import tpu_sc as plsc

N_NODES = 10000
D_FEAT = 256
N_EDGES = 160000

NC = 2
NS = 16
DH = D_FEAT // NC
E_PER_TILE = N_EDGES // NS
B = 80
NCHUNK = E_PER_TILE // B
NBUF = 4
LA = 2
NBUFI = 8
LAI = 6
ROWS_PER_TILE = 624
ROWS_TAIL = N_NODES - NS * ROWS_PER_TILE
ROWS_Z = 208


def _mp_body(x, ei_flat, zeros, out,
             acc, sidx, didx, rows, gsem, ssem, isem_s, isem_d):
    c = lax.axis_index("c")
    s = lax.axis_index("s")
    col0 = c * DH

    base_e = s * E_PER_TILE

    def start_idx(j):
        bi = lax.rem(j, NBUFI)
        st = base_e + j * B
        pltpu.async_copy(ei_flat.at[pl.ds(N_EDGES + st, B)], sidx.at[bi],
                         isem_s.at[bi])
        pltpu.async_copy(ei_flat.at[pl.ds(st, B)], didx.at[bi],
                         isem_d.at[bi])

    def wait_idx(j):
        bi = lax.rem(j, NBUFI)
        st = base_e + j * B
        pltpu.make_async_copy(ei_flat.at[pl.ds(N_EDGES + st, B)],
                              sidx.at[bi], isem_s.at[bi]).wait()
        pltpu.make_async_copy(ei_flat.at[pl.ds(st, B)], didx.at[bi],
                              isem_d.at[bi]).wait()

    def start_gather(j, b):
        bi = lax.rem(j, NBUFI)
        pltpu.async_copy(x.at[sidx.at[bi], pl.ds(col0, DH)], rows.at[b],
                         gsem.at[b])

    def wait_gather(j, b):
        bi = lax.rem(j, NBUFI)
        pltpu.make_async_copy(x.at[sidx.at[bi], pl.ds(col0, DH)],
                              rows.at[b], gsem.at[b]).wait()

    def start_scatter(j, b):
        bi = lax.rem(j, NBUFI)
        pltpu.async_copy(rows.at[b], acc.at[didx.at[bi]], ssem.at[b],
                         add=True)

    def wait_scatter(j, b):
        bi = lax.rem(j, NBUFI)
        pltpu.make_async_copy(rows.at[b], acc.at[didx.at[bi]],
                              ssem.at[b]).wait()

    for k in range(LAI):
        start_idx(k)
    for k in range(LA):
        wait_idx(k)
        start_gather(k, k)

    row0 = s * ROWS_PER_TILE
    for r in range(ROWS_PER_TILE // ROWS_Z):
        pltpu.sync_copy(zeros, acc.at[pl.ds(row0 + r * ROWS_Z, ROWS_Z)])

    @pl.when(s == 0)
    def _zero_tail():
        pltpu.sync_copy(zeros.at[pl.ds(0, ROWS_TAIL)],
                        acc.at[pl.ds(NS * ROWS_PER_TILE, ROWS_TAIL)])

    plsc.subcore_barrier()

    def chunk(j, carry):
        b = lax.rem(j, NBUF)
        wait_gather(j, b)
        start_scatter(j, b)

        @pl.when(j >= LA)
        def _drain():
            wait_scatter(j - LA, lax.rem(j - LA, NBUF))

        @pl.when(j + LAI < NCHUNK)
        def _pf_idx():
            start_idx(j + LAI)

        @pl.when(j + LA < NCHUNK)
        def _pf_gather():
            wait_idx(j + LA)
            start_gather(j + LA, lax.rem(j + LA, NBUF))

        return carry

    lax.fori_loop(0, NCHUNK, chunk, 0)

    for j in range(NCHUNK - LA, NCHUNK):
        wait_scatter(j, j % NBUF)

    plsc.subcore_barrier()
    pltpu.sync_copy(acc.at[pl.ds(row0, ROWS_PER_TILE)],
                    out.at[pl.ds(row0, ROWS_PER_TILE), pl.ds(col0, DH)])

    @pl.when(s == 0)
    def _out_tail():
        pltpu.sync_copy(
            acc.at[pl.ds(NS * ROWS_PER_TILE, ROWS_TAIL)],
            out.at[pl.ds(NS * ROWS_PER_TILE, ROWS_TAIL), pl.ds(col0, DH)])


_mp_call = functools.partial(
    pl.kernel,
    out_type=jax.ShapeDtypeStruct((N_NODES, D_FEAT), jnp.float32),
    mesh=plsc.VectorSubcoreMesh(core_axis_name="c", subcore_axis_name="s",
                                num_cores=NC, num_subcores=NS),
    scratch_types=[
        pltpu.VMEM_SHARED((N_NODES, DH), jnp.float32),
        pltpu.VMEM((NBUFI, B), jnp.int32),
        pltpu.VMEM((NBUFI, B), jnp.int32),
        pltpu.VMEM((NBUF, B, DH), jnp.float32),
        pltpu.SemaphoreType.DMA((NBUF,)),
        pltpu.SemaphoreType.DMA((NBUF,)),
        pltpu.SemaphoreType.DMA((NBUFI,)),
        pltpu.SemaphoreType.DMA((NBUFI,)),
    ],
)(_mp_body)


def kernel(x, edge_index):
    ei_flat = edge_index.astype(jnp.int32).reshape(2 * N_EDGES)
    zeros = jnp.zeros((ROWS_Z, DH), jnp.float32)
    return _mp_call(x, ei_flat, zeros)
